# Initial kernel scaffold; baseline (speedup 1.0000x reference)
import jax
import jax.numpy as jnp
from jax.experimental import pallas as pl


def kernel(fieldlist, textlist, htmllist, html_edge_src, html_edge_type, html_edge_dst, ht_html, ht_text, sequencelist, w_html, w_text, position_embedding, htmledge_embedding, dense_kernel, dense_bias):
    raise NotImplementedError("write your pallas kernel here")


# speedup vs baseline: 3.3089x; 3.3089x over previous
import functools

import jax
import jax.numpy as jnp
from jax import lax
from jax.experimental import pallas as pl
from jax.experimental.pallas import tpu as pltpu
from jax.experimental.pallas import tpu_sc as plsc

B, T, H, F, S, NE, NHT = 32, 512, 128, 16, 8, 64, 128
WORD, TAG, HID, WIDTH, MAXPOS = 21128, 512, 384, 16, 4
D_IN = 768

_BM = 512
_NBLK = (WORD + _BM - 1) // _BM


def _proj_body(w_ref, k_ref, b_ref, o_ref):
    o_ref[...] = (
        jnp.dot(w_ref[...], k_ref[...], preferred_element_type=jnp.float32)
        + b_ref[...]
    )


def _project_table(w_text, dense_kernel, dense_bias):
    return pl.pallas_call(
        _proj_body,
        grid=(_NBLK,),
        in_specs=[
            pl.BlockSpec((_BM, D_IN), lambda i: (i, 0)),
            pl.BlockSpec((D_IN, HID), lambda i: (0, 0)),
            pl.BlockSpec((1, HID), lambda i: (0, 0)),
        ],
        out_specs=pl.BlockSpec((_BM, HID), lambda i: (i, 0)),
        out_shape=jax.ShapeDtypeStruct((WORD, HID), jnp.float32),
    )(w_text, dense_kernel, dense_bias.reshape(1, HID))


_TEXT_CHUNK = 128


def _make_gather():
    mesh = plsc.VectorSubcoreMesh(core_axis_name="c", subcore_axis_name="s")

    @functools.partial(
        pl.kernel,
        out_type=(
            jax.ShapeDtypeStruct((B * T, HID), jnp.float32),
            jax.ShapeDtypeStruct((B * F, D_IN), jnp.float32),
            jax.ShapeDtypeStruct((B * H, HID), jnp.float32),
        ),
        mesh=mesh,
        scratch_types=(
            pltpu.VMEM((T // _TEXT_CHUNK, _TEXT_CHUNK), jnp.int32),
            pltpu.VMEM((F,), jnp.int32),
            pltpu.VMEM((H,), jnp.int32),
            pltpu.VMEM((_TEXT_CHUNK, HID), jnp.float32),
            pltpu.VMEM((F, D_IN), jnp.float32),
            pltpu.VMEM((H, HID), jnp.float32),
            pltpu.SemaphoreType.DMA,
        ),
    )
    def gather(p_hbm, wt_hbm, wh_hbm, tidx_hbm, fidx_hbm, hidx_hbm,
               text_out, field_out, html_out,
               tidx_v, fidx_v, hidx_v, trows_v, frows_v, hrows_v, sem):
        wid = lax.axis_index("s") * 2 + lax.axis_index("c")
        pltpu.sync_copy(fidx_hbm.at[wid], fidx_v)
        pltpu.async_copy(wt_hbm.at[fidx_v], frows_v, sem).wait()
        pltpu.sync_copy(frows_v, field_out.at[pl.ds(wid * F, F)])
        pltpu.sync_copy(hidx_hbm.at[wid], hidx_v)
        pltpu.async_copy(wh_hbm.at[hidx_v], hrows_v, sem).wait()
        pltpu.sync_copy(hrows_v, html_out.at[pl.ds(wid * H, H)])
        pltpu.sync_copy(tidx_hbm.at[wid], tidx_v)
        for j in range(T // _TEXT_CHUNK):
            pltpu.async_copy(p_hbm.at[tidx_v.at[j]], trows_v, sem).wait()
            pltpu.sync_copy(
                trows_v,
                text_out.at[pl.ds(wid * T + j * _TEXT_CHUNK, _TEXT_CHUNK)],
            )

    return gather


_gather_cache = []


def _gather_fn(*args):
    if not _gather_cache:
        _gather_cache.append(_make_gather())
    return _gather_cache[0](*args)


def _t2t_body(seq_ref, out_ref):
    kk = lax.broadcasted_iota(jnp.int32, (T, 1), 0)
    ll = lax.broadcasted_iota(jnp.int32, (1, T), 1)
    segk = jnp.zeros((T, 1), jnp.int32)
    segl = jnp.zeros((1, T), jnp.int32)
    for j in range(S):
        sj = seq_ref[0, 0, j]
        segk = segk + (sj < kk).astype(jnp.int32)
        segl = segl + (sj < ll).astype(jnp.int32)
    last = seq_ref[0, 0, S - 1]
    validk = (segk >= 1) & (kk <= last)
    validl = (segl >= 1) & (ll <= last)
    ck = jnp.where(validk, segk * 1024 + kk, -1000000)
    cl = jnp.where(validl, segl * 1024 + ll, 1000000)
    d = jnp.abs(ck - cl)
    out_ref[0] = jnp.where(d <= WIDTH, 0.0, 1.0).astype(jnp.float32)


def _t2t_mask(sequencelist):
    return pl.pallas_call(
        _t2t_body,
        grid=(B,),
        in_specs=[
            pl.BlockSpec((1, 1, S), lambda b: (b, 0, 0),
                         memory_space=pltpu.SMEM)
        ],
        out_specs=pl.BlockSpec((1, T, T), lambda b: (b, 0, 0)),
        out_shape=jax.ShapeDtypeStruct((B, T, T), jnp.float32),
    )(sequencelist.reshape(B, 1, S))


def _masks_body(hth_ref, htt_ref, src_ref, dst_ref, typ_ref, hl_ref,
                h2t_ref, h2h_ref, t2h_ref):
    iota_h = lax.broadcasted_iota(jnp.int32, (H, 1), 0)
    iota_t = lax.broadcasted_iota(jnp.int32, (T, 1), 0)
    a = (iota_h == hth_ref[0]).astype(jnp.float32)
    bt = (iota_t == htt_ref[0]).astype(jnp.float32)
    c = lax.dot_general(a, bt, (((1,), (1,)), ((), ())),
                        preferred_element_type=jnp.float32)
    h2t_ref[0] = jnp.where(c > 0.5, 0.0, 1.0)

    h2h_ref[0] = jnp.zeros((H, H), jnp.int32)
    col = lax.broadcasted_iota(jnp.int32, (1, H), 1)

    def body(j, carry):
        s = src_ref[0, 0, j]
        d = dst_ref[0, 0, j]
        t = typ_ref[0, 0, j]
        row = h2h_ref[0, pl.ds(s, 1), :]
        h2h_ref[0, pl.ds(s, 1), :] = jnp.where(col == d, t, row)
        return carry

    lax.fori_loop(0, NE, body, 0)

    t2h_ref[0] = hl_ref[0] == 0


def _masks(ht_html, ht_text, src, dst, typ, htmllist):
    h2t, h2h, t2h = pl.pallas_call(
        _masks_body,
        grid=(B,),
        in_specs=[
            pl.BlockSpec((1, 1, NHT), lambda b: (b, 0, 0)),
            pl.BlockSpec((1, 1, NHT), lambda b: (b, 0, 0)),
            pl.BlockSpec((1, 1, NE), lambda b: (b, 0, 0),
                         memory_space=pltpu.SMEM),
            pl.BlockSpec((1, 1, NE), lambda b: (b, 0, 0),
                         memory_space=pltpu.SMEM),
            pl.BlockSpec((1, 1, NE), lambda b: (b, 0, 0),
                         memory_space=pltpu.SMEM),
            pl.BlockSpec((1, 1, H), lambda b: (b, 0, 0)),
        ],
        out_specs=[
            pl.BlockSpec((1, H, T), lambda b: (b, 0, 0)),
            pl.BlockSpec((1, H, H), lambda b: (b, 0, 0)),
            pl.BlockSpec((1, 1, H), lambda b: (b, 0, 0)),
        ],
        out_shape=[
            jax.ShapeDtypeStruct((B, H, T), jnp.float32),
            jax.ShapeDtypeStruct((B, H, H), jnp.int32),
            jax.ShapeDtypeStruct((B, 1, H), jnp.bool_),
        ],
    )(ht_html.reshape(B, 1, NHT), ht_text.reshape(B, 1, NHT),
      src.reshape(B, 1, NE), dst.reshape(B, 1, NE), typ.reshape(B, 1, NE),
      htmllist.reshape(B, 1, H))
    return h2t, h2h, t2h.reshape(B, H)


def kernel(fieldlist, textlist, htmllist, html_edge_src, html_edge_type,
           html_edge_dst, ht_html, ht_text, sequencelist, w_html, w_text,
           position_embedding, htmledge_embedding, dense_kernel, dense_bias):
    i32 = jnp.int32
    tidx = textlist.astype(i32).reshape(B, T // _TEXT_CHUNK, _TEXT_CHUNK)
    fidx = fieldlist.astype(i32)
    hidx = htmllist.astype(i32)

    p_table = _project_table(w_text, dense_kernel, dense_bias)
    text_rows, field_rows, html_rows = _gather_fn(
        p_table, w_text, w_html, tidx, fidx, hidx)

    field_embeds = field_rows.reshape(B, F, D_IN)
    text_embeds = text_rows.reshape(B, T, HID)
    html_embeds = html_rows.reshape(B, H, HID)

    t2t = _t2t_mask(sequencelist.astype(i32))
    h2t, h2h, t2h = _masks(
        ht_html.astype(i32), ht_text.astype(i32),
        html_edge_src.astype(i32), html_edge_dst.astype(i32),
        html_edge_type.astype(i32), htmllist.astype(i32))

    htmledge_complete = jnp.concatenate(
        [jnp.ones((1, HID), jnp.float32), htmledge_embedding], axis=0)

    return (field_embeds, text_embeds, html_embeds, t2t, h2h, h2t, t2h,
            position_embedding, htmledge_complete)
